# baseline (speedup 1.0000x reference)
import jax
import jax.numpy as jnp
from jax.experimental import pallas as pl
from jax.experimental.pallas import tpu as pltpu

_R = 64
_G = 2
_RG = _R // _G
_C = 1_000_000
_BC = 8192
_CB = (_C + _BC - 1) // _BC

_NEG_INF = float("-inf")


def _pass1_kernel(d_ref, m_out, s_out, m_s, s_s):
    j = pl.program_id(1)
    x = d_ref[...]
    col = jax.lax.broadcasted_iota(jnp.int32, (_RG, _BC), 1) + j * _BC
    x = jnp.where(col < _C, x, jnp.float32(_NEG_INF))
    bm = jnp.max(x, axis=1, keepdims=True)

    @pl.when(j == 0)
    def _init():
        m_s[...] = bm
        s_s[...] = jnp.sum(jnp.exp(x - bm), axis=1, keepdims=True)

    @pl.when(j > 0)
    def _acc():
        m_prev = m_s[...]
        m_new = jnp.maximum(m_prev, bm)
        s_s[...] = (s_s[...] * jnp.exp(m_prev - m_new)
                    + jnp.sum(jnp.exp(x - m_new), axis=1, keepdims=True))
        m_s[...] = m_new

    @pl.when(j == _CB - 1)
    def _fin():
        m_out[...] = m_s[...]
        s_out[...] = s_s[...]


def _pass2_kernel(d_ref, k_ref, m_ref, s_ref, o_ref, idx_ref, gval_s, gidx_s):
    j = pl.program_id(1)
    x = d_ref[...]
    msk = k_ref[...]
    m = m_ref[...]
    inv_s = 1.0 / s_ref[...]

    p = jnp.exp(x - m) * inv_s
    o_ref[...] = p

    col = jax.lax.broadcasted_iota(jnp.int32, (_RG, _BC), 1) + j * _BC
    valid = jnp.logical_and(col < _C, jnp.logical_not(msk))
    q = jnp.where(valid, p + jnp.float32(1e-5), jnp.float32(0.0))
    bq = jnp.max(q, axis=1, keepdims=True)
    cand = jnp.where(q == bq, col, jnp.int32(_C))
    bi = jnp.min(cand, axis=1, keepdims=True)

    @pl.when(j == 0)
    def _init():
        gval_s[...] = jnp.full((_RG, 1), -1.0, jnp.float32)
        gidx_s[...] = jnp.zeros((_RG, 1), jnp.int32)

    upd = bq > gval_s[...]
    gval_s[...] = jnp.where(upd, bq, gval_s[...])
    gidx_s[...] = jnp.where(upd, bi, gidx_s[...])

    @pl.when(j == _CB - 1)
    def _fin():
        idx_ref[...] = gidx_s[...]


def kernel(dist, mask):
    m, s = pl.pallas_call(
        _pass1_kernel,
        grid=(_G, _CB),
        in_specs=[pl.BlockSpec((_RG, _BC), lambda g, j: (g, j))],
        out_specs=[
            pl.BlockSpec((_RG, 1), lambda g, j: (g, 0)),
            pl.BlockSpec((_RG, 1), lambda g, j: (g, 0)),
        ],
        out_shape=[
            jax.ShapeDtypeStruct((_R, 1), jnp.float32),
            jax.ShapeDtypeStruct((_R, 1), jnp.float32),
        ],
        scratch_shapes=[
            pltpu.VMEM((_RG, 1), jnp.float32),
            pltpu.VMEM((_RG, 1), jnp.float32),
        ],
        compiler_params=pltpu.CompilerParams(
            dimension_semantics=("parallel", "arbitrary"),
        ),
    )(dist)

    out, idx = pl.pallas_call(
        _pass2_kernel,
        grid=(_G, _CB),
        in_specs=[
            pl.BlockSpec((_RG, _BC), lambda g, j: (g, j)),
            pl.BlockSpec((_RG, _BC), lambda g, j: (g, j)),
            pl.BlockSpec((_RG, 1), lambda g, j: (g, 0)),
            pl.BlockSpec((_RG, 1), lambda g, j: (g, 0)),
        ],
        out_specs=[
            pl.BlockSpec((_RG, _BC), lambda g, j: (g, j)),
            pl.BlockSpec((_RG, 1), lambda g, j: (g, 0)),
        ],
        out_shape=[
            jax.ShapeDtypeStruct((_R, _C), jnp.float32),
            jax.ShapeDtypeStruct((_R, 1), jnp.int32),
        ],
        scratch_shapes=[
            pltpu.VMEM((_RG, 1), jnp.float32),
            pltpu.VMEM((_RG, 1), jnp.int32),
        ],
        compiler_params=pltpu.CompilerParams(
            dimension_semantics=("parallel", "arbitrary"),
        ),
    )(dist, mask, m, s)
    return out, idx


# speedup vs baseline: 2.0955x; 2.0955x over previous
import jax
import jax.numpy as jnp
from jax.experimental import pallas as pl
from jax.experimental.pallas import tpu as pltpu

_R = 64
_C = 1_000_000
_BC = 32768
_CB = (_C + _BC - 1) // _BC

_NEG_INF = float("-inf")


def _pass1_kernel(d_ref, m_out, s_out, m_s, s_s):
    j = pl.program_id(0)
    x = d_ref[...]
    col = jax.lax.broadcasted_iota(jnp.int32, (_R, _BC), 1) + j * _BC
    x = jnp.where(col < _C, x, jnp.float32(_NEG_INF))
    bm = jnp.max(x, axis=1, keepdims=True)

    @pl.when(j == 0)
    def _init():
        m_s[...] = bm
        s_s[...] = jnp.sum(jnp.exp(x - bm), axis=1, keepdims=True)

    @pl.when(j > 0)
    def _acc():
        m_prev = m_s[...]
        m_new = jnp.maximum(m_prev, bm)
        s_s[...] = (s_s[...] * jnp.exp(m_prev - m_new)
                    + jnp.sum(jnp.exp(x - m_new), axis=1, keepdims=True))
        m_s[...] = m_new

    @pl.when(j == _CB - 1)
    def _fin():
        m_out[...] = m_s[...]
        s_out[...] = s_s[...]


def _pass2_kernel(d_ref, k_ref, m_ref, s_ref, o_ref, idx_ref, gval_s, gidx_s):
    j = pl.program_id(0)
    x = d_ref[...]
    m = m_ref[...]
    inv_s = 1.0 / s_ref[...]

    p = jnp.exp(x - m) * inv_s
    o_ref[...] = p

    col = jax.lax.broadcasted_iota(jnp.int32, (_R, _BC), 1) + j * _BC
    valid = jnp.logical_and(col < _C, k_ref[...].astype(jnp.int32) == 0)
    q = jnp.where(valid, p + jnp.float32(1e-5), jnp.float32(0.0))
    bq = jnp.max(q, axis=1, keepdims=True)
    cand = jnp.where(q == bq, col, jnp.int32(_C))
    bi = jnp.min(cand, axis=1, keepdims=True)

    @pl.when(j == 0)
    def _init():
        gval_s[...] = jnp.full((_R, 1), -1.0, jnp.float32)
        gidx_s[...] = jnp.zeros((_R, 1), jnp.int32)

    upd = bq > gval_s[...]
    gval_s[...] = jnp.where(upd, bq, gval_s[...])
    gidx_s[...] = jnp.where(upd, bi, gidx_s[...])

    @pl.when(j == _CB - 1)
    def _fin():
        idx_ref[...] = gidx_s[...]


def kernel(dist, mask):
    mask_i8 = mask.view(jnp.int8)
    m, s = pl.pallas_call(
        _pass1_kernel,
        grid=(_CB,),
        in_specs=[pl.BlockSpec((_R, _BC), lambda j: (0, j))],
        out_specs=[
            pl.BlockSpec((_R, 1), lambda j: (0, 0)),
            pl.BlockSpec((_R, 1), lambda j: (0, 0)),
        ],
        out_shape=[
            jax.ShapeDtypeStruct((_R, 1), jnp.float32),
            jax.ShapeDtypeStruct((_R, 1), jnp.float32),
        ],
        scratch_shapes=[
            pltpu.VMEM((_R, 1), jnp.float32),
            pltpu.VMEM((_R, 1), jnp.float32),
        ],
        compiler_params=pltpu.CompilerParams(
            dimension_semantics=("arbitrary",),
        ),
    )(dist)

    out, idx = pl.pallas_call(
        _pass2_kernel,
        grid=(_CB,),
        in_specs=[
            pl.BlockSpec((_R, _BC), lambda j: (0, j)),
            pl.BlockSpec((_R, _BC), lambda j: (0, j)),
            pl.BlockSpec((_R, 1), lambda j: (0, 0)),
            pl.BlockSpec((_R, 1), lambda j: (0, 0)),
        ],
        out_specs=[
            pl.BlockSpec((_R, _BC), lambda j: (0, j)),
            pl.BlockSpec((_R, 1), lambda j: (0, 0)),
        ],
        out_shape=[
            jax.ShapeDtypeStruct((_R, _C), jnp.float32),
            jax.ShapeDtypeStruct((_R, 1), jnp.int32),
        ],
        scratch_shapes=[
            pltpu.VMEM((_R, 1), jnp.float32),
            pltpu.VMEM((_R, 1), jnp.int32),
        ],
        compiler_params=pltpu.CompilerParams(
            dimension_semantics=("arbitrary",),
        ),
    )(dist, mask_i8, m, s)
    return out, idx


# speedup vs baseline: 2.1039x; 1.0040x over previous
import jax
import jax.numpy as jnp
from jax.experimental import pallas as pl
from jax.experimental.pallas import tpu as pltpu

_R = 64
_C = 1_000_000
_BC = 32768
_CB = (_C + _BC - 1) // _BC


def _pass1_kernel(d_ref, m_out, s_out, m_s, s_s):
    j = pl.program_id(0)
    x = d_ref[...]
    col = jax.lax.broadcasted_iota(jnp.int32, (_R, _BC), 1) + j * _BC
    x = jnp.where(col < _C, x, jnp.float32(float("-inf")))
    bm = jnp.max(x, axis=1, keepdims=True)

    first = j == 0
    m_prev = jnp.where(first, bm, m_s[...])
    s_prev = jnp.where(first, jnp.float32(0.0), s_s[...])
    m_new = jnp.maximum(m_prev, bm)
    s_s[...] = (s_prev * jnp.exp(m_prev - m_new)
                + jnp.sum(jnp.exp(x - m_new), axis=1, keepdims=True))
    m_s[...] = m_new

    @pl.when(j == _CB - 1)
    def _fin():
        m_out[...] = m_s[...]
        s_out[...] = s_s[...]


def _pass2_kernel(d_ref, k_ref, m_ref, s_ref, o_ref, idx_ref, gval_s, gidx_s):
    j = pl.program_id(0)
    x = d_ref[...]
    m = m_ref[...]
    inv_s = 1.0 / s_ref[...]

    p = jnp.exp(x - m) * inv_s
    o_ref[...] = p

    col = jax.lax.broadcasted_iota(jnp.int32, (_R, _BC), 1) + j * _BC
    notmask = jnp.logical_and(col < _C, k_ref[...].astype(jnp.int32) == 0)
    q = jnp.where(notmask, p + jnp.float32(1e-5), jnp.float32(0.0))
    bq = jnp.max(q, axis=1, keepdims=True)
    cand = jnp.where(q == bq, col, jnp.int32(_C))
    bi = jnp.min(cand, axis=1, keepdims=True)

    first = j == 0
    gval_prev = jnp.where(first, jnp.float32(-1.0), gval_s[...])
    gidx_prev = jnp.where(first, jnp.int32(0), gidx_s[...])
    upd = bq > gval_prev
    gval_s[...] = jnp.where(upd, bq, gval_prev)
    gidx_s[...] = jnp.where(upd, bi, gidx_prev)

    @pl.when(j == _CB - 1)
    def _fin():
        idx_ref[...] = gidx_s[...]


def kernel(dist, mask):
    mask_i8 = mask.view(jnp.int8)
    m, s = pl.pallas_call(
        _pass1_kernel,
        grid=(_CB,),
        in_specs=[pl.BlockSpec((_R, _BC), lambda j: (0, j))],
        out_specs=[
            pl.BlockSpec((_R, 1), lambda j: (0, 0)),
            pl.BlockSpec((_R, 1), lambda j: (0, 0)),
        ],
        out_shape=[
            jax.ShapeDtypeStruct((_R, 1), jnp.float32),
            jax.ShapeDtypeStruct((_R, 1), jnp.float32),
        ],
        scratch_shapes=[
            pltpu.VMEM((_R, 1), jnp.float32),
            pltpu.VMEM((_R, 1), jnp.float32),
        ],
        compiler_params=pltpu.CompilerParams(
            dimension_semantics=("arbitrary",),
        ),
    )(dist)

    out, idx = pl.pallas_call(
        _pass2_kernel,
        grid=(_CB,),
        in_specs=[
            pl.BlockSpec((_R, _BC), lambda j: (0, j)),
            pl.BlockSpec((_R, _BC), lambda j: (0, j)),
            pl.BlockSpec((_R, 1), lambda j: (0, 0)),
            pl.BlockSpec((_R, 1), lambda j: (0, 0)),
        ],
        out_specs=[
            pl.BlockSpec((_R, _BC), lambda j: (0, j)),
            pl.BlockSpec((_R, 1), lambda j: (0, 0)),
        ],
        out_shape=[
            jax.ShapeDtypeStruct((_R, _C), jnp.float32),
            jax.ShapeDtypeStruct((_R, 1), jnp.int32),
        ],
        scratch_shapes=[
            pltpu.VMEM((_R, 1), jnp.float32),
            pltpu.VMEM((_R, 1), jnp.int32),
        ],
        compiler_params=pltpu.CompilerParams(
            dimension_semantics=("arbitrary",),
        ),
    )(dist, mask_i8, m, s)
    return out, idx


# speedup vs baseline: 2.1156x; 1.0056x over previous
import jax
import jax.numpy as jnp
from jax.experimental import pallas as pl
from jax.experimental.pallas import tpu as pltpu

_R = 64
_C = 1_000_000
_BC1 = 65536
_CB1 = (_C + _BC1 - 1) // _BC1
_BC = 32768
_CB = (_C + _BC - 1) // _BC


def _pass1_kernel(d_ref, m_out, s_out, m_s, s_s):
    j = pl.program_id(0)
    x = d_ref[...]
    col = jax.lax.broadcasted_iota(jnp.int32, (_R, _BC1), 1) + j * _BC1
    x = jnp.where(col < _C, x, jnp.float32(float("-inf")))
    bm = jnp.max(x, axis=1, keepdims=True)

    first = j == 0
    m_prev = jnp.where(first, bm, m_s[...])
    s_prev = jnp.where(first, jnp.float32(0.0), s_s[...])
    m_new = jnp.maximum(m_prev, bm)
    s_s[...] = (s_prev * jnp.exp(m_prev - m_new)
                + jnp.sum(jnp.exp(x - m_new), axis=1, keepdims=True))
    m_s[...] = m_new

    @pl.when(j == _CB1 - 1)
    def _fin():
        m_out[...] = m_s[...]
        s_out[...] = s_s[...]


def _pass2_kernel(d_ref, k_ref, m_ref, s_ref, o_ref, idx_ref, gval_s, gidx_s):
    j = pl.program_id(0)
    x = d_ref[...]
    m = m_ref[...]
    inv_s = 1.0 / s_ref[...]

    p = jnp.exp(x - m) * inv_s
    o_ref[...] = p

    col = jax.lax.broadcasted_iota(jnp.int32, (_R, _BC), 1) + j * _BC
    notmask = jnp.logical_and(col < _C, k_ref[...].astype(jnp.int32) == 0)
    q = jnp.where(notmask, p + jnp.float32(1e-5), jnp.float32(0.0))
    bq = jnp.max(q, axis=1, keepdims=True)
    cand = jnp.where(q == bq, col, jnp.int32(_C))
    bi = jnp.min(cand, axis=1, keepdims=True)

    first = j == 0
    gval_prev = jnp.where(first, jnp.float32(-1.0), gval_s[...])
    gidx_prev = jnp.where(first, jnp.int32(0), gidx_s[...])
    upd = bq > gval_prev
    gval_s[...] = jnp.where(upd, bq, gval_prev)
    gidx_s[...] = jnp.where(upd, bi, gidx_prev)

    @pl.when(j == _CB - 1)
    def _fin():
        idx_ref[...] = gidx_s[...]


def kernel(dist, mask):
    mask_i8 = mask.view(jnp.int8)
    m, s = pl.pallas_call(
        _pass1_kernel,
        grid=(_CB1,),
        in_specs=[pl.BlockSpec((_R, _BC1), lambda j: (0, j))],
        out_specs=[
            pl.BlockSpec((_R, 1), lambda j: (0, 0)),
            pl.BlockSpec((_R, 1), lambda j: (0, 0)),
        ],
        out_shape=[
            jax.ShapeDtypeStruct((_R, 1), jnp.float32),
            jax.ShapeDtypeStruct((_R, 1), jnp.float32),
        ],
        scratch_shapes=[
            pltpu.VMEM((_R, 1), jnp.float32),
            pltpu.VMEM((_R, 1), jnp.float32),
        ],
        compiler_params=pltpu.CompilerParams(
            dimension_semantics=("arbitrary",),
        ),
    )(dist)

    out, idx = pl.pallas_call(
        _pass2_kernel,
        grid=(_CB,),
        in_specs=[
            pl.BlockSpec((_R, _BC), lambda j: (0, j)),
            pl.BlockSpec((_R, _BC), lambda j: (0, j)),
            pl.BlockSpec((_R, 1), lambda j: (0, 0)),
            pl.BlockSpec((_R, 1), lambda j: (0, 0)),
        ],
        out_specs=[
            pl.BlockSpec((_R, _BC), lambda j: (0, j)),
            pl.BlockSpec((_R, 1), lambda j: (0, 0)),
        ],
        out_shape=[
            jax.ShapeDtypeStruct((_R, _C), jnp.float32),
            jax.ShapeDtypeStruct((_R, 1), jnp.int32),
        ],
        scratch_shapes=[
            pltpu.VMEM((_R, 1), jnp.float32),
            pltpu.VMEM((_R, 1), jnp.int32),
        ],
        compiler_params=pltpu.CompilerParams(
            dimension_semantics=("arbitrary",),
        ),
    )(dist, mask_i8, m, s)
    return out, idx
